# baseline (speedup 1.0000x reference)
import jax
import jax.numpy as jnp
from jax import lax
from jax.experimental import pallas as pl
from jax.experimental.pallas import tpu as pltpu
from jax.experimental.pallas import tpu_sc as plsc

DIM = 16
BATCH = 16384
NC = 2
NS = 16
L = 16
NW = NC * NS
BPW = BATCH // NW
IDX_CHUNK = 128
NCHUNK = (2 * BPW) // IDX_CHUNK


def _loss_body(x_hbm, deg_hbm, embs_hbm, out_hbm, idx_v, rows_v, deg_v,
               res_v, sem):
    wid = lax.axis_index("s") * NC + lax.axis_index("c")
    pltpu.sync_copy(x_hbm.at[wid], idx_v)
    pltpu.sync_copy(deg_hbm.at[wid], deg_v)
    copies = [
        pltpu.async_copy(embs_hbm.at[idx_v.at[k]],
                         rows_v.at[pl.ds(k * IDX_CHUNK, IDX_CHUNK)], sem)
        for k in range(NCHUNK)
    ]
    for cp in copies:
        cp.wait()

    iota = lax.iota(jnp.int32, L)

    def sqrt16(s):
        i = plsc.bitcast(s, jnp.int32)
        i = jnp.int32(0x5F3759DF) - (i >> 1)
        y = plsc.bitcast(i, jnp.float32)
        for _ in range(3):
            y = y * (1.5 - 0.5 * s * y * y)
        return jnp.where(s > 0.0, s * y, 0.0)

    def body(g, acc):
        r0 = (g * L + iota) * 2
        r1 = r0 + 1
        s = jnp.zeros((L,), jnp.float32)
        for d in range(DIM):
            col = jnp.full((L,), d, jnp.int32)
            a = plsc.load_gather(rows_v, [r0, col])
            b = plsc.load_gather(rows_v, [r1, col])
            df = a - b
            s = s + df * df
        dist = sqrt16(s)
        deg = deg_v[pl.ds(g * L, L)]
        rate = jnp.exp(-dist)
        return acc + deg * dist + rate

    acc = lax.fori_loop(0, BPW // L, body, jnp.zeros((L,), jnp.float32))
    res_v[...] = jnp.full((L,), jnp.sum(acc), jnp.float32)
    pltpu.sync_copy(res_v, out_hbm.at[wid])


def kernel(x, degrees, embs):
    xr = x.astype(jnp.int32).reshape(NW, NCHUNK, IDX_CHUNK)
    dr = degrees.reshape(NW, BPW)
    mesh = plsc.VectorSubcoreMesh(core_axis_name="c", subcore_axis_name="s")
    out = pl.kernel(
        _loss_body,
        mesh=mesh,
        out_type=jax.ShapeDtypeStruct((NW, L), jnp.float32),
        scratch_types=[
            pltpu.VMEM((NCHUNK, IDX_CHUNK), jnp.int32),
            pltpu.VMEM((2 * BPW, DIM), jnp.float32),
            pltpu.VMEM((BPW,), jnp.float32),
            pltpu.VMEM((L,), jnp.float32),
            pltpu.SemaphoreType.DMA,
        ],
        compiler_params=pltpu.CompilerParams(needs_layout_passes=False,
                                             use_tc_tiling_on_sc=False),
    )(xr, dr, embs)
    return jnp.sum(out[:, 0])
